# Initial kernel scaffold; baseline (speedup 1.0000x reference)
import jax
import jax.numpy as jnp
from jax.experimental import pallas as pl


def kernel(x, W, W1a, b1a, W1b, b1b):
    raise NotImplementedError("write your pallas kernel here")


# speedup vs baseline: 2.4903x; 2.4903x over previous
import itertools

import numpy as np
import jax
import jax.numpy as jnp
from jax.experimental import pallas as pl

_N_CLUSTERS = 16
_DIM = 32
_N_POINTS = 16384
_PAIRS = np.array(list(itertools.combinations(range(_N_CLUSTERS), 2)), dtype=np.int32)
_TRIPLES = np.array(list(itertools.combinations(range(_N_CLUSTERS), 3)), dtype=np.int32)
_NC = _N_CLUSTERS + len(_PAIRS) + len(_TRIPLES)
_NC_PAD = 768
_BR = 2048
_GRID = _N_POINTS // _BR


def _prep_body(w_ref, wp0_ref, wp1_ref, wt0_ref, wt1_ref, wt2_ref,
               w1a_ref, b1a_ref, w1b_ref, b1b_ref, cnt_ref, t_ref):
    w1aT = w1a_ref[...].T
    w1bT = w1b_ref[...].T
    b1a = b1a_ref[...]
    b1b = b1b_ref[...]

    def g_net(x1, x2):
        return ((jnp.dot(x1, w1aT, preferred_element_type=jnp.float32) + b1a)
                + (jnp.dot(x2, w1aT, preferred_element_type=jnp.float32) + b1a)
                + (jnp.dot(x1 * x2, w1bT, preferred_element_type=jnp.float32) + b1b))

    emb2 = g_net(wp0_ref[...], wp1_ref[...])
    tmp = g_net(wt0_ref[...], wt1_ref[...])
    emb3 = g_net(tmp, wt2_ref[...])
    cents = jnp.concatenate([w_ref[...], emb2, emb3], axis=0)
    nrm = jnp.sqrt(jnp.sum(cents * cents, axis=1, keepdims=True))
    cn = cents / jnp.maximum(nrm, 1e-12)
    t = jnp.sum(cn * cn, axis=1)
    cnt = jnp.concatenate(
        [cn.T, jnp.zeros((_DIM, _NC_PAD - _NC), jnp.float32)], axis=1)
    cnt_ref[...] = cnt
    tpad = jnp.concatenate([t, jnp.full((_NC_PAD - _NC,), jnp.inf, jnp.float32)])
    t_ref[...] = jnp.broadcast_to(tpad[None, :], (8, _NC_PAD))


def _main_body(x_ref, cnt_ref, t_ref, assign_ref, acc_ref):
    i = pl.program_id(0)
    xb = x_ref[...]
    nrm = jnp.sqrt(jnp.sum(xb * xb, axis=1, keepdims=True))
    xn = xb / jnp.maximum(nrm, 1e-12)
    san = jnp.sum(xn * xn, axis=1, keepdims=True)
    s = jnp.dot(xn, cnt_ref[...], preferred_element_type=jnp.float32)
    d2 = (san + t_ref[0:1, :]) - 2.0 * s
    dd = jnp.maximum(d2, 0.0)
    m = jnp.min(dd, axis=1, keepdims=True)
    col = jax.lax.broadcasted_iota(jnp.int32, (_BR, _NC_PAD), 1)
    idx = jnp.min(jnp.where(dd == m, col, jnp.int32(2147483647)), axis=1)
    assign_ref[...] = idx.reshape(_BR // 128, 128)

    @pl.when(i == 0)
    def _():
        acc_ref[...] = jnp.zeros_like(acc_ref)

    acc_ref[...] += m.reshape(_BR // 128, 128)


def kernel(x, W, W1a, b1a, W1b, b1b):
    wp0 = jnp.take(W, _PAIRS[:, 0], axis=0)
    wp1 = jnp.take(W, _PAIRS[:, 1], axis=0)
    wt0 = jnp.take(W, _TRIPLES[:, 0], axis=0)
    wt1 = jnp.take(W, _TRIPLES[:, 1], axis=0)
    wt2 = jnp.take(W, _TRIPLES[:, 2], axis=0)

    cnt, tvec = pl.pallas_call(
        _prep_body,
        out_shape=(
            jax.ShapeDtypeStruct((_DIM, _NC_PAD), jnp.float32),
            jax.ShapeDtypeStruct((8, _NC_PAD), jnp.float32),
        ),
    )(W, wp0, wp1, wt0, wt1, wt2, W1a, b1a[None, :], W1b, b1b[None, :])

    rows = _BR // 128
    assign2d, acc = pl.pallas_call(
        _main_body,
        grid=(_GRID,),
        in_specs=[
            pl.BlockSpec((_BR, _DIM), lambda i: (i, 0)),
            pl.BlockSpec((_DIM, _NC_PAD), lambda i: (0, 0)),
            pl.BlockSpec((8, _NC_PAD), lambda i: (0, 0)),
        ],
        out_specs=(
            pl.BlockSpec((rows, 128), lambda i: (i, 0)),
            pl.BlockSpec((rows, 128), lambda i: (0, 0)),
        ),
        out_shape=(
            jax.ShapeDtypeStruct((_GRID * rows, 128), jnp.int32),
            jax.ShapeDtypeStruct((rows, 128), jnp.float32),
        ),
    )(x, cnt, tvec)

    assignment = assign2d.reshape(_N_POINTS)
    dists = jnp.sqrt(jnp.sum(acc))
    return (dists, assignment)


# speedup vs baseline: 3.4347x; 1.3792x over previous
import itertools

import numpy as np
import jax
import jax.numpy as jnp
from jax.experimental import pallas as pl
from jax.experimental.pallas import tpu as pltpu

_N_CLUSTERS = 16
_DIM = 32
_N_POINTS = 16384
_PAIRS = np.array(list(itertools.combinations(range(_N_CLUSTERS), 2)), dtype=np.int32)
_TRIPLES = np.array(list(itertools.combinations(range(_N_CLUSTERS), 3)), dtype=np.int32)
_NP_ = len(_PAIRS)
_NT = len(_TRIPLES)
_NC = _N_CLUSTERS + _NP_ + _NT
_NC_PAD = 768
_BR = 2048
_GRID = _N_POINTS // _BR

_P0 = np.zeros((128, 1), np.int32); _P0[:_NP_, 0] = _PAIRS[:, 0]
_P1 = np.zeros((128, 1), np.int32); _P1[:_NP_, 0] = _PAIRS[:, 1]
_T0 = _TRIPLES[:, 0:1].copy()
_T1 = _TRIPLES[:, 1:2].copy()
_T2 = _TRIPLES[:, 2:3].copy()


def _gather16(w, idx_col, nrows):
    out = jnp.zeros((nrows, _DIM), jnp.float32)
    for k in range(_N_CLUSTERS):
        row = jnp.broadcast_to(w[k:k + 1, :], (nrows, _DIM))
        out = jnp.where(idx_col == k, row, out)
    return out


def _body(w_ref, w1a_ref, b1a_ref, w1b_ref, b1b_ref,
          p0_ref, p1_ref, t0_ref, t1_ref, t2_ref, x_ref,
          assign_ref, dists_ref, cnt_s, t_s, acc_s):
    i = pl.program_id(0)

    @pl.when(i == 0)
    def _prep():
        w = w_ref[...]
        w1aT = w1a_ref[...].T
        w1bT = w1b_ref[...].T
        b1a = b1a_ref[...]
        b1b = b1b_ref[...]

        def g_net(x1, x2):
            return ((jnp.dot(x1, w1aT, preferred_element_type=jnp.float32) + b1a)
                    + (jnp.dot(x2, w1aT, preferred_element_type=jnp.float32) + b1a)
                    + (jnp.dot(x1 * x2, w1bT, preferred_element_type=jnp.float32) + b1b))

        wp0 = _gather16(w, p0_ref[...], 128)[:_NP_]
        wp1 = _gather16(w, p1_ref[...], 128)[:_NP_]
        wt0 = _gather16(w, t0_ref[...], _NT)
        wt1 = _gather16(w, t1_ref[...], _NT)
        wt2 = _gather16(w, t2_ref[...], _NT)
        emb2 = g_net(wp0, wp1)
        tmp = g_net(wt0, wt1)
        emb3 = g_net(tmp, wt2)
        cents = jnp.concatenate([w, emb2, emb3], axis=0)
        nrm = jnp.sqrt(jnp.sum(cents * cents, axis=1, keepdims=True))
        cn = cents / jnp.maximum(nrm, 1e-12)
        t = jnp.sum(cn * cn, axis=1)
        cnt_s[...] = jnp.concatenate(
            [cn.T, jnp.zeros((_DIM, _NC_PAD - _NC), jnp.float32)], axis=1)
        tpad = jnp.concatenate([t, jnp.full((_NC_PAD - _NC,), jnp.inf, jnp.float32)])
        t_s[...] = jnp.broadcast_to(tpad[None, :], (8, _NC_PAD))
        acc_s[...] = jnp.zeros_like(acc_s)

    xb = x_ref[...]
    nrm = jnp.sqrt(jnp.sum(xb * xb, axis=1, keepdims=True))
    xn = xb / jnp.maximum(nrm, 1e-12)
    san = jnp.sum(xn * xn, axis=1, keepdims=True)
    s = jnp.dot(xn, cnt_s[...], preferred_element_type=jnp.float32)
    d2 = (san + t_s[0:1, :]) - 2.0 * s
    dd = jnp.maximum(d2, 0.0)
    m = jnp.min(dd, axis=1, keepdims=True)
    col = jax.lax.broadcasted_iota(jnp.int32, (_BR, _NC_PAD), 1)
    idx = jnp.min(jnp.where(dd == m, col, jnp.int32(2147483647)), axis=1)
    assign_ref[...] = idx.reshape(_BR // 128, 128)
    acc_s[...] += m.reshape(_BR // 128, 128)

    @pl.when(i == _GRID - 1)
    def _fin():
        dists_ref[...] = jnp.sqrt(jnp.sum(acc_s[...])).reshape(1, 1)


def kernel(x, W, W1a, b1a, W1b, b1b):
    rows = _BR // 128
    assign2d, dists = pl.pallas_call(
        _body,
        grid=(_GRID,),
        in_specs=[
            pl.BlockSpec((_N_CLUSTERS, _DIM), lambda i: (0, 0)),
            pl.BlockSpec((_DIM, _DIM), lambda i: (0, 0)),
            pl.BlockSpec((1, _DIM), lambda i: (0, 0)),
            pl.BlockSpec((_DIM, _DIM), lambda i: (0, 0)),
            pl.BlockSpec((1, _DIM), lambda i: (0, 0)),
            pl.BlockSpec((128, 1), lambda i: (0, 0)),
            pl.BlockSpec((128, 1), lambda i: (0, 0)),
            pl.BlockSpec((_NT, 1), lambda i: (0, 0)),
            pl.BlockSpec((_NT, 1), lambda i: (0, 0)),
            pl.BlockSpec((_NT, 1), lambda i: (0, 0)),
            pl.BlockSpec((_BR, _DIM), lambda i: (i, 0)),
        ],
        out_specs=(
            pl.BlockSpec((rows, 128), lambda i: (i, 0)),
            pl.BlockSpec((1, 1), lambda i: (0, 0)),
        ),
        out_shape=(
            jax.ShapeDtypeStruct((_GRID * rows, 128), jnp.int32),
            jax.ShapeDtypeStruct((1, 1), jnp.float32),
        ),
        scratch_shapes=[
            pltpu.VMEM((_DIM, _NC_PAD), jnp.float32),
            pltpu.VMEM((8, _NC_PAD), jnp.float32),
            pltpu.VMEM((rows, 128), jnp.float32),
        ],
    )(W, W1a, b1a[None, :], W1b, b1b[None, :],
      jnp.asarray(_P0), jnp.asarray(_P1), jnp.asarray(_T0),
      jnp.asarray(_T1), jnp.asarray(_T2), x)

    return (dists.reshape(()), assign2d.reshape(_N_POINTS))


# speedup vs baseline: 3.5400x; 1.0307x over previous
import itertools

import numpy as np
import jax
import jax.numpy as jnp
from jax.experimental import pallas as pl
from jax.experimental.pallas import tpu as pltpu

_N_CLUSTERS = 16
_DIM = 32
_N_POINTS = 16384
_PAIRS = np.array(list(itertools.combinations(range(_N_CLUSTERS), 2)), dtype=np.int32)
_TRIPLES = np.array(list(itertools.combinations(range(_N_CLUSTERS), 3)), dtype=np.int32)
_NP_ = len(_PAIRS)
_NT = len(_TRIPLES)
_NC = _N_CLUSTERS + _NP_ + _NT
_NC_PAD = 768
_BR = 4096
_GRID = _N_POINTS // _BR

_P0 = np.zeros((128, 1), np.int32); _P0[:_NP_, 0] = _PAIRS[:, 0]
_P1 = np.zeros((128, 1), np.int32); _P1[:_NP_, 0] = _PAIRS[:, 1]
_T0 = _TRIPLES[:, 0:1].copy()
_T1 = _TRIPLES[:, 1:2].copy()
_T2 = _TRIPLES[:, 2:3].copy()


def _gather16(w, idx_col, nrows):
    out = jnp.zeros((nrows, _DIM), jnp.float32)
    for k in range(_N_CLUSTERS):
        row = jnp.broadcast_to(w[k:k + 1, :], (nrows, _DIM))
        out = jnp.where(idx_col == k, row, out)
    return out


def _body(w_ref, w1a_ref, b1a_ref, w1b_ref, b1b_ref,
          p0_ref, p1_ref, t0_ref, t1_ref, t2_ref, x_ref,
          assign_ref, dists_ref, cnt_s, t_s, acc_s):
    i = pl.program_id(0)

    @pl.when(i == 0)
    def _prep():
        w = w_ref[...]
        w1aT = w1a_ref[...].T
        w1bT = w1b_ref[...].T
        b1a = b1a_ref[...]
        b1b = b1b_ref[...]

        def g_net(x1, x2):
            return ((jnp.dot(x1, w1aT, preferred_element_type=jnp.float32) + b1a)
                    + (jnp.dot(x2, w1aT, preferred_element_type=jnp.float32) + b1a)
                    + (jnp.dot(x1 * x2, w1bT, preferred_element_type=jnp.float32) + b1b))

        wp0 = _gather16(w, p0_ref[...], 128)[:_NP_]
        wp1 = _gather16(w, p1_ref[...], 128)[:_NP_]
        wt0 = _gather16(w, t0_ref[...], _NT)
        wt1 = _gather16(w, t1_ref[...], _NT)
        wt2 = _gather16(w, t2_ref[...], _NT)
        emb2 = g_net(wp0, wp1)
        tmp = g_net(wt0, wt1)
        emb3 = g_net(tmp, wt2)
        cents = jnp.concatenate([w, emb2, emb3], axis=0)
        nrm = jnp.sqrt(jnp.sum(cents * cents, axis=1, keepdims=True))
        cn = cents / jnp.maximum(nrm, 1e-12)
        t = jnp.sum(cn * cn, axis=1)
        cnt_s[...] = jnp.concatenate(
            [cn.T, jnp.zeros((_DIM, _NC_PAD - _NC), jnp.float32)], axis=1)
        tpad = jnp.concatenate([t, jnp.full((_NC_PAD - _NC,), jnp.inf, jnp.float32)])
        t_s[...] = jnp.broadcast_to(tpad[None, :], (8, _NC_PAD))
        acc_s[...] = jnp.zeros_like(acc_s)

    xb = x_ref[...]
    nrm = jnp.sqrt(jnp.sum(xb * xb, axis=1, keepdims=True))
    xn = xb / jnp.maximum(nrm, 1e-12)
    san = jnp.sum(xn * xn, axis=1, keepdims=True)
    s = jnp.dot(xn, cnt_s[...], preferred_element_type=jnp.float32)
    d2 = (san + t_s[0:1, :]) - 2.0 * s
    m = jnp.maximum(jnp.min(d2, axis=1, keepdims=True), 0.0)
    col = jax.lax.broadcasted_iota(jnp.int32, (_BR, _NC_PAD), 1)
    idx = jnp.min(jnp.where(d2 <= m, col, jnp.int32(2147483647)), axis=1)
    assign_ref[...] = idx.reshape(_BR // 128, 128)
    acc_s[...] += m.reshape(_BR // 128, 128)

    @pl.when(i == _GRID - 1)
    def _fin():
        dists_ref[...] = jnp.sqrt(jnp.sum(acc_s[...])).reshape(1, 1)


def kernel(x, W, W1a, b1a, W1b, b1b):
    rows = _BR // 128
    assign2d, dists = pl.pallas_call(
        _body,
        grid=(_GRID,),
        in_specs=[
            pl.BlockSpec((_N_CLUSTERS, _DIM), lambda i: (0, 0)),
            pl.BlockSpec((_DIM, _DIM), lambda i: (0, 0)),
            pl.BlockSpec((1, _DIM), lambda i: (0, 0)),
            pl.BlockSpec((_DIM, _DIM), lambda i: (0, 0)),
            pl.BlockSpec((1, _DIM), lambda i: (0, 0)),
            pl.BlockSpec((128, 1), lambda i: (0, 0)),
            pl.BlockSpec((128, 1), lambda i: (0, 0)),
            pl.BlockSpec((_NT, 1), lambda i: (0, 0)),
            pl.BlockSpec((_NT, 1), lambda i: (0, 0)),
            pl.BlockSpec((_NT, 1), lambda i: (0, 0)),
            pl.BlockSpec((_BR, _DIM), lambda i: (i, 0)),
        ],
        out_specs=(
            pl.BlockSpec((rows, 128), lambda i: (i, 0)),
            pl.BlockSpec((1, 1), lambda i: (0, 0)),
        ),
        out_shape=(
            jax.ShapeDtypeStruct((_GRID * rows, 128), jnp.int32),
            jax.ShapeDtypeStruct((1, 1), jnp.float32),
        ),
        scratch_shapes=[
            pltpu.VMEM((_DIM, _NC_PAD), jnp.float32),
            pltpu.VMEM((8, _NC_PAD), jnp.float32),
            pltpu.VMEM((rows, 128), jnp.float32),
        ],
    )(W, W1a, b1a[None, :], W1b, b1b[None, :],
      jnp.asarray(_P0), jnp.asarray(_P1), jnp.asarray(_T0),
      jnp.asarray(_T1), jnp.asarray(_T2), x)

    return (dists.reshape(()), assign2d.reshape(_N_POINTS))
